# Initial kernel scaffold; baseline (speedup 1.0000x reference)
import jax
import jax.numpy as jnp
from jax.experimental import pallas as pl


def kernel(x, edge_index, edge_attr, batch, W1, b1, W2, b2, Wl1, bl1, Wl2, bl2):
    raise NotImplementedError("write your pallas kernel here")


# speedup vs baseline: 7.0221x; 7.0221x over previous
import functools
import jax
import jax.numpy as jnp
from jax import lax
from jax.experimental import pallas as pl
from jax.experimental.pallas import tpu as pltpu
from jax.experimental.pallas import tpu_sc as plsc

N = 50000
E = 800000
G = 64
NP = 50176
EPAD = 819200
NW = 32
EP_T = EPAD // NW
CH4 = 6400
CH6 = 8192
VEC = 16
C1 = 104
C2 = 200

def _wid():
    return lax.axis_index("s") * 2 + lax.axis_index("c")


def _sc_mesh():
    return plsc.VectorSubcoreMesh(core_axis_name="c", subcore_axis_name="s")


def _k1_body(col_hbm, w_hbm, zeros_hbm, out_hbm, colb, wb, table):
    wid = _wid()
    base = wid * EP_T
    pltpu.sync_copy(zeros_hbm, table)

    def chunk(j, _):
        pltpu.sync_copy(col_hbm.at[pl.ds(base + j * CH4, CH4)], colb)
        pltpu.sync_copy(w_hbm.at[pl.ds(base + j * CH4, CH4)], wb)

        def body(i, _):
            c16 = colb[pl.ds(i * VEC, VEC)]
            w16 = wb[pl.ds(i * VEC, VEC)]
            plsc.addupdate_scatter(table, [c16], w16)
            return 0

        lax.fori_loop(0, CH4 // VEC, body, 0)
        return 0

    lax.fori_loop(0, EP_T // CH4, chunk, 0)
    pltpu.sync_copy(table, out_hbm.at[wid])


def _k2_body(part_ref, dinv_ref, self_ref):
    deg = jnp.sum(part_ref[...], axis=0, keepdims=True) + 1.0
    dinv_ref[...] = lax.rsqrt(deg)
    self_ref[...] = 1.0 / deg


def _k2_dinv(part):
    nb = NP // 512
    return pl.pallas_call(
        _k2_body,
        grid=(nb,),
        in_specs=[pl.BlockSpec((NW, 512), lambda i: (0, i))],
        out_specs=[pl.BlockSpec((1, 512), lambda i: (0, i)),
                   pl.BlockSpec((1, 512), lambda i: (0, i))],
        out_shape=[jax.ShapeDtypeStruct((1, NP), jnp.float32),
                   jax.ShapeDtypeStruct((1, NP), jnp.float32)],
    )(part)


def _k3_body(row_hbm, col_hbm, w_hbm, dinv_hbm, out_hbm,
             rowb, colb, wb, normb, dinvv):
    wid = _wid()
    base = wid * EP_T
    pltpu.sync_copy(dinv_hbm, dinvv)

    def chunk(j, _):
        off = base + j * CH4
        pltpu.sync_copy(row_hbm.at[pl.ds(off, CH4)], rowb)
        pltpu.sync_copy(col_hbm.at[pl.ds(off, CH4)], colb)
        pltpu.sync_copy(w_hbm.at[pl.ds(off, CH4)], wb)

        def body(i, _):
            r16 = rowb[pl.ds(i * VEC, VEC)]
            c16 = colb[pl.ds(i * VEC, VEC)]
            w16 = wb[pl.ds(i * VEC, VEC)]
            dr = plsc.load_gather(dinvv, [r16])
            dc = plsc.load_gather(dinvv, [c16])
            normb[pl.ds(i * VEC, VEC)] = dr * w16 * dc
            return 0

        lax.fori_loop(0, CH4 // VEC, body, 0)
        pltpu.sync_copy(normb, out_hbm.at[pl.ds(off, CH4)])
        return 0

    lax.fori_loop(0, EP_T // CH4, chunk, 0)


def _k4_body(row_hbm, col_hbm, norm_hbm, xt_hbm, zeros_hbm, out_hbm,
             rowb, colb, normb, schan, table):
    wid = _wid()
    base = wid * EP_T
    for ch in range(3):
        pltpu.sync_copy(xt_hbm.at[ch], schan)
        pltpu.sync_copy(zeros_hbm, table)

        def chunk(j, _):
            off = base + j * CH4
            pltpu.sync_copy(row_hbm.at[pl.ds(off, CH4)], rowb)
            pltpu.sync_copy(col_hbm.at[pl.ds(off, CH4)], colb)
            pltpu.sync_copy(norm_hbm.at[pl.ds(off, CH4)], normb)

            def body(i, _):
                r16 = rowb[pl.ds(i * VEC, VEC)]
                c16 = colb[pl.ds(i * VEC, VEC)]
                n16 = normb[pl.ds(i * VEC, VEC)]
                g = plsc.load_gather(schan, [r16])
                plsc.addupdate_scatter(table, [c16], g * n16)
                return 0

            lax.fori_loop(0, CH4 // VEC, body, 0)
            return 0

        lax.fori_loop(0, EP_T // CH4, chunk, 0)
        pltpu.sync_copy(table, out_hbm.at[ch, wid])


def _k5_body(part_ref, xt_ref, self_ref, w1t_ref, b1_ref, out_ref):
    p1 = jnp.sum(part_ref[...], axis=1) + xt_ref[...] * self_ref[...]
    h = jnp.dot(w1t_ref[...], p1, preferred_element_type=jnp.float32)
    h = h + b1_ref[...]
    out_ref[...] = h * jax.nn.sigmoid(h)


def _k5_h1(part1, xt, selfn, w1t, b1c):
    nb = NP // 512
    return pl.pallas_call(
        _k5_body,
        grid=(nb,),
        in_specs=[pl.BlockSpec((3, NW, 512), lambda i: (0, 0, i)),
                  pl.BlockSpec((3, 512), lambda i: (0, i)),
                  pl.BlockSpec((1, 512), lambda i: (0, i)),
                  pl.BlockSpec((C1, 3), lambda i: (0, 0)),
                  pl.BlockSpec((C1, 1), lambda i: (0, 0))],
        out_specs=pl.BlockSpec((C1, 512), lambda i: (0, i)),
        out_shape=jax.ShapeDtypeStruct((C1, NP), jnp.float32),
    )(part1, xt, selfn, w1t, b1c)


def _k6_body(row_hbm, col_hbm, norm_hbm, h1t_hbm, zeros_hbm, out_hbm,
             rowb, colb, normb, schan, table):
    wid = _wid()

    def run_channel(c):
        pltpu.sync_copy(h1t_hbm.at[c], schan)

        def chunk(j, _):
            off = j * CH6
            pltpu.sync_copy(row_hbm.at[pl.ds(off, CH6)], rowb)
            pltpu.sync_copy(col_hbm.at[pl.ds(off, CH6)], colb)
            pltpu.sync_copy(norm_hbm.at[pl.ds(off, CH6)], normb)

            def body(i, _):
                r16 = rowb[pl.ds(i * VEC, VEC)]
                c16 = colb[pl.ds(i * VEC, VEC)]
                n16 = normb[pl.ds(i * VEC, VEC)]
                g = plsc.load_gather(schan, [r16])
                plsc.addupdate_scatter(table, [c16], g * n16)
                return 0

            lax.fori_loop(0, CH6 // VEC, body, 0)
            return 0

        lax.fori_loop(0, EPAD // CH6, chunk, 0)

    for k in range(3):
        c = k * NW + wid
        pltpu.sync_copy(zeros_hbm, table)
        run_channel(c)
        pltpu.sync_copy(table, out_hbm.at[c])

    c = 3 * NW + wid

    @pl.when(wid < 8)
    def _():
        pltpu.sync_copy(zeros_hbm, table)

    @pl.when(wid < 4)
    def _():
        run_channel(c)

    @pl.when(wid < 8)
    def _():
        pltpu.sync_copy(table, out_hbm.at[c])


def _k7_body(p2_ref, h1_ref, self_ref, batch_ref, w2t_ref, b2_ref,
             wl1t_ref, bl1_ref, wl2t_ref, bl2_ref, out_ref,
             pool_ref, cnt_ref):
    i = pl.program_id(0)

    @pl.when(i == 0)
    def _():
        pool_ref[...] = jnp.zeros_like(pool_ref)
        cnt_ref[...] = jnp.zeros_like(cnt_ref)

    p2 = p2_ref[...] + h1_ref[...] * self_ref[...]
    h = jnp.dot(w2t_ref[...], p2, preferred_element_type=jnp.float32)
    h = h + b2_ref[...]
    h2 = h * jax.nn.sigmoid(h)
    onehot = (batch_ref[...].reshape(512, 1) ==
              lax.broadcasted_iota(jnp.int32, (1, G), 1)).astype(jnp.float32)
    pool_ref[...] += jnp.dot(h2, onehot, preferred_element_type=jnp.float32)
    cnt_ref[...] += jnp.sum(onehot, axis=0, keepdims=True)

    @pl.when(i == pl.num_programs(0) - 1)
    def _():
        pooled = pool_ref[...] / jnp.maximum(cnt_ref[...], 1.0)
        t = jnp.dot(wl1t_ref[...], pooled, preferred_element_type=jnp.float32)
        t = t + bl1_ref[...]
        t = t * jax.nn.sigmoid(t)
        o = jnp.dot(wl2t_ref[...], t, preferred_element_type=jnp.float32)
        out_ref[...] = o + bl2_ref[...]


def _k7_head(p2t, h1t, selfn, batchp, w2t, b2c, wl1t, bl1c, wl2t, bl2c):
    nb = NP // 512
    return pl.pallas_call(
        _k7_body,
        grid=(nb,),
        in_specs=[pl.BlockSpec((C1, 512), lambda i: (0, i)),
                  pl.BlockSpec((C1, 512), lambda i: (0, i)),
                  pl.BlockSpec((1, 512), lambda i: (0, i)),
                  pl.BlockSpec((1, 512), lambda i: (0, i)),
                  pl.BlockSpec((C2, C1), lambda i: (0, 0)),
                  pl.BlockSpec((C2, 1), lambda i: (0, 0)),
                  pl.BlockSpec((100, C2), lambda i: (0, 0)),
                  pl.BlockSpec((100, 1), lambda i: (0, 0)),
                  pl.BlockSpec((1, 100), lambda i: (0, 0)),
                  pl.BlockSpec((1, 1), lambda i: (0, 0))],
        out_specs=pl.BlockSpec((1, G), lambda i: (0, 0)),
        out_shape=jax.ShapeDtypeStruct((1, G), jnp.float32),
        scratch_shapes=[pltpu.VMEM((C2, G), jnp.float32),
                        pltpu.VMEM((1, G), jnp.float32)],
    )(p2t, h1t, selfn, batchp, w2t, b2c, wl1t, bl1c, wl2t, bl2c)


@functools.cache
def _sc_kernels():
    mesh = _sc_mesh()
    cp = pltpu.CompilerParams(needs_layout_passes=False,
                              use_tc_tiling_on_sc=False)
    k1 = functools.partial(
        pl.kernel, mesh=mesh, compiler_params=cp,
        out_type=jax.ShapeDtypeStruct((NW, NP), jnp.float32),
        scratch_types=[
            pltpu.VMEM((CH4,), jnp.int32),
            pltpu.VMEM((CH4,), jnp.float32),
            pltpu.VMEM((NP,), jnp.float32),
        ],
    )(_k1_body)
    k3 = functools.partial(
        pl.kernel, mesh=mesh, compiler_params=cp,
        out_type=jax.ShapeDtypeStruct((EPAD,), jnp.float32),
        scratch_types=[
            pltpu.VMEM((CH4,), jnp.int32),
            pltpu.VMEM((CH4,), jnp.int32),
            pltpu.VMEM((CH4,), jnp.float32),
            pltpu.VMEM((CH4,), jnp.float32),
            pltpu.VMEM((NP,), jnp.float32),
        ],
    )(_k3_body)
    k4 = functools.partial(
        pl.kernel, mesh=mesh, compiler_params=cp,
        out_type=jax.ShapeDtypeStruct((3, NW, NP), jnp.float32),
        scratch_types=[
            pltpu.VMEM((CH4,), jnp.int32),
            pltpu.VMEM((CH4,), jnp.int32),
            pltpu.VMEM((CH4,), jnp.float32),
            pltpu.VMEM((NP,), jnp.float32),
            pltpu.VMEM((NP,), jnp.float32),
        ],
    )(_k4_body)
    k6 = functools.partial(
        pl.kernel, mesh=mesh, compiler_params=cp,
        out_type=jax.ShapeDtypeStruct((C1, NP), jnp.float32),
        scratch_types=[
            pltpu.VMEM((CH6,), jnp.int32),
            pltpu.VMEM((CH6,), jnp.int32),
            pltpu.VMEM((CH6,), jnp.float32),
            pltpu.VMEM((NP,), jnp.float32),
            pltpu.VMEM((NP,), jnp.float32),
        ],
    )(_k6_body)
    return k1, k3, k4, k6


def kernel(x, edge_index, edge_attr, batch, W1, b1, W2, b2, Wl1, bl1, Wl2, bl2):
    _k1_deg, _k3_norm, _k4_prop1, _k6_prop2 = _sc_kernels()
    f32 = jnp.float32
    pad_e = EPAD - E
    row = jnp.concatenate([edge_index[0], jnp.zeros((pad_e,), jnp.int32)])
    col = jnp.concatenate([edge_index[1], jnp.zeros((pad_e,), jnp.int32)])
    w = jnp.concatenate([edge_attr.astype(f32), jnp.zeros((pad_e,), f32)])

    xt = jnp.pad(x.astype(f32).T, ((0, 0), (0, NP - N)))
    batchp = jnp.pad(batch, (0, NP - N), constant_values=G).reshape(1, NP)
    zerosN = jnp.zeros((NP,), f32)

    w1t = jnp.pad(W1.T, ((0, C1 - 100), (0, 0)))
    b1c = jnp.pad(b1.reshape(100, 1), ((0, C1 - 100), (0, 0)))
    w2t = jnp.pad(W2.T, ((0, 0), (0, C1 - 100)))
    b2c = b2.reshape(C2, 1)
    wl1t = Wl1.T
    bl1c = bl1.reshape(100, 1)
    wl2t = Wl2.T
    bl2c = bl2.reshape(1, 1)

    part_deg = _k1_deg(col, w, zerosN)
    dinv2d, selfn = _k2_dinv(part_deg)
    norm = _k3_norm(row, col, w, dinv2d.reshape(NP))
    part1 = _k4_prop1(row, col, norm, xt, zerosN)
    h1t = _k5_h1(part1, xt, selfn, w1t, b1c)
    p2t = _k6_prop2(row, col, norm, h1t, zerosN)
    out = _k7_head(p2t, h1t, selfn, batchp, w2t, b2c,
                   wl1t, bl1c, wl2t, bl2c)
    return out[0]


# speedup vs baseline: 10.9998x; 1.5665x over previous
import functools
import jax
import jax.numpy as jnp
from jax import lax
from jax.experimental import pallas as pl
from jax.experimental.pallas import tpu as pltpu
from jax.experimental.pallas import tpu_sc as plsc

N = 50000
E = 800000
G = 64
NP = 50176
EPAD = 819200
NW = 32
EP_T = EPAD // NW
CH4 = 6400
CH6 = 8192
VEC = 16
C1 = 104
C2 = 200

def _wid():
    return lax.axis_index("s") * 2 + lax.axis_index("c")


def _sc_mesh():
    return plsc.VectorSubcoreMesh(core_axis_name="c", subcore_axis_name="s")


def _k1_body(col_hbm, w_hbm, zeros_hbm, out_hbm, colb, wb, table):
    wid = _wid()
    base = wid * EP_T
    pltpu.sync_copy(zeros_hbm, table)

    def chunk(j, _):
        pltpu.sync_copy(col_hbm.at[pl.ds(base + j * CH4, CH4)], colb)
        pltpu.sync_copy(w_hbm.at[pl.ds(base + j * CH4, CH4)], wb)

        def body(i, _):
            c16 = colb[pl.ds(i * VEC, VEC)]
            w16 = wb[pl.ds(i * VEC, VEC)]
            plsc.addupdate_scatter(table, [c16], w16)
            return 0

        lax.fori_loop(0, CH4 // VEC, body, 0)
        return 0

    lax.fori_loop(0, EP_T // CH4, chunk, 0)
    pltpu.sync_copy(table, out_hbm.at[wid])


def _k2_body(part_ref, dinv_ref, self_ref):
    deg = jnp.sum(part_ref[...], axis=0, keepdims=True) + 1.0
    dinv_ref[...] = lax.rsqrt(deg)
    self_ref[...] = 1.0 / deg


def _k2_dinv(part):
    nb = NP // 512
    return pl.pallas_call(
        _k2_body,
        grid=(nb,),
        in_specs=[pl.BlockSpec((NW, 512), lambda i: (0, i))],
        out_specs=[pl.BlockSpec((1, 512), lambda i: (0, i)),
                   pl.BlockSpec((1, 512), lambda i: (0, i))],
        out_shape=[jax.ShapeDtypeStruct((1, NP), jnp.float32),
                   jax.ShapeDtypeStruct((1, NP), jnp.float32)],
    )(part)


def _k3_body(row_hbm, col_hbm, w_hbm, dinv_hbm, out_hbm,
             rowb, colb, wb, normb, dinvv):
    wid = _wid()
    base = wid * EP_T
    pltpu.sync_copy(dinv_hbm, dinvv)

    def chunk(j, _):
        off = base + j * CH4
        pltpu.sync_copy(row_hbm.at[pl.ds(off, CH4)], rowb)
        pltpu.sync_copy(col_hbm.at[pl.ds(off, CH4)], colb)
        pltpu.sync_copy(w_hbm.at[pl.ds(off, CH4)], wb)

        def body(i, _):
            r16 = rowb[pl.ds(i * VEC, VEC)]
            c16 = colb[pl.ds(i * VEC, VEC)]
            w16 = wb[pl.ds(i * VEC, VEC)]
            dr = plsc.load_gather(dinvv, [r16])
            dc = plsc.load_gather(dinvv, [c16])
            normb[pl.ds(i * VEC, VEC)] = dr * w16 * dc
            return 0

        lax.fori_loop(0, CH4 // VEC, body, 0)
        pltpu.sync_copy(normb, out_hbm.at[pl.ds(off, CH4)])
        return 0

    lax.fori_loop(0, EP_T // CH4, chunk, 0)


def _k4_body(row_hbm, col_hbm, norm_hbm, xt_hbm, zeros_hbm, out_hbm,
             rowb, colb, normb, schan, table):
    wid = _wid()
    base = wid * EP_T
    for ch in range(3):
        pltpu.sync_copy(xt_hbm.at[ch], schan)
        pltpu.sync_copy(zeros_hbm, table)

        def chunk(j, _):
            off = base + j * CH4
            pltpu.sync_copy(row_hbm.at[pl.ds(off, CH4)], rowb)
            pltpu.sync_copy(col_hbm.at[pl.ds(off, CH4)], colb)
            pltpu.sync_copy(norm_hbm.at[pl.ds(off, CH4)], normb)

            def body(i, _):
                r16 = rowb[pl.ds(i * VEC, VEC)]
                c16 = colb[pl.ds(i * VEC, VEC)]
                n16 = normb[pl.ds(i * VEC, VEC)]
                g = plsc.load_gather(schan, [r16])
                plsc.addupdate_scatter(table, [c16], g * n16)
                return 0

            lax.fori_loop(0, CH4 // VEC, body, 0)
            return 0

        lax.fori_loop(0, EP_T // CH4, chunk, 0)
        pltpu.sync_copy(table, out_hbm.at[ch, wid])


def _k5_body(part_ref, xt_ref, self_ref, w1t_ref, b1_ref, out_ref):
    p1 = jnp.sum(part_ref[...], axis=1) + xt_ref[...] * self_ref[...]
    h = jnp.dot(w1t_ref[...], p1, preferred_element_type=jnp.float32)
    h = h + b1_ref[...]
    out_ref[...] = h * jax.nn.sigmoid(h)


def _k5_h1(part1, xt, selfn, w1t, b1c):
    nb = NP // 512
    return pl.pallas_call(
        _k5_body,
        grid=(nb,),
        in_specs=[pl.BlockSpec((3, NW, 512), lambda i: (0, 0, i)),
                  pl.BlockSpec((3, 512), lambda i: (0, i)),
                  pl.BlockSpec((1, 512), lambda i: (0, i)),
                  pl.BlockSpec((C1, 3), lambda i: (0, 0)),
                  pl.BlockSpec((C1, 1), lambda i: (0, 0))],
        out_specs=pl.BlockSpec((C1, 512), lambda i: (0, i)),
        out_shape=jax.ShapeDtypeStruct((C1, NP), jnp.float32),
    )(part1, xt, selfn, w1t, b1c)


def _k6_body(row_hbm, col_hbm, norm_hbm, h1t_hbm, zeros_hbm, out_hbm,
             rowb, colb, normb, schan, table):
    wid = _wid()

    def run_channel(c):
        pltpu.sync_copy(h1t_hbm.at[c], schan)

        def chunk(j, _):
            off = j * CH6
            pltpu.sync_copy(row_hbm.at[pl.ds(off, CH6)], rowb)
            pltpu.sync_copy(col_hbm.at[pl.ds(off, CH6)], colb)
            pltpu.sync_copy(norm_hbm.at[pl.ds(off, CH6)], normb)

            @plsc.parallel_loop(0, CH6 // VEC, unroll=8)
            def _(i):
                r16 = rowb[pl.ds(i * VEC, VEC)]
                c16 = colb[pl.ds(i * VEC, VEC)]
                n16 = normb[pl.ds(i * VEC, VEC)]
                g = plsc.load_gather(schan, [r16])
                plsc.addupdate_scatter(table, [c16], g * n16)

            return 0

        lax.fori_loop(0, EPAD // CH6, chunk, 0)

    for k in range(3):
        c = k * NW + wid
        pltpu.sync_copy(zeros_hbm, table)
        run_channel(c)
        pltpu.sync_copy(table, out_hbm.at[c])

    c = 3 * NW + wid

    @pl.when(wid < 8)
    def _():
        pltpu.sync_copy(zeros_hbm, table)

    @pl.when(wid < 4)
    def _():
        run_channel(c)

    @pl.when(wid < 8)
    def _():
        pltpu.sync_copy(table, out_hbm.at[c])


def _k7_body(p2_ref, h1_ref, self_ref, batch_ref, w2t_ref, b2_ref,
             wl1t_ref, bl1_ref, wl2t_ref, bl2_ref, out_ref,
             pool_ref, cnt_ref):
    i = pl.program_id(0)

    @pl.when(i == 0)
    def _():
        pool_ref[...] = jnp.zeros_like(pool_ref)
        cnt_ref[...] = jnp.zeros_like(cnt_ref)

    p2 = p2_ref[...] + h1_ref[...] * self_ref[...]
    h = jnp.dot(w2t_ref[...], p2, preferred_element_type=jnp.float32)
    h = h + b2_ref[...]
    h2 = h * jax.nn.sigmoid(h)
    onehot = (batch_ref[...].reshape(512, 1) ==
              lax.broadcasted_iota(jnp.int32, (1, G), 1)).astype(jnp.float32)
    pool_ref[...] += jnp.dot(h2, onehot, preferred_element_type=jnp.float32)
    cnt_ref[...] += jnp.sum(onehot, axis=0, keepdims=True)

    @pl.when(i == pl.num_programs(0) - 1)
    def _():
        pooled = pool_ref[...] / jnp.maximum(cnt_ref[...], 1.0)
        t = jnp.dot(wl1t_ref[...], pooled, preferred_element_type=jnp.float32)
        t = t + bl1_ref[...]
        t = t * jax.nn.sigmoid(t)
        o = jnp.dot(wl2t_ref[...], t, preferred_element_type=jnp.float32)
        out_ref[...] = o + bl2_ref[...]


def _k7_head(p2t, h1t, selfn, batchp, w2t, b2c, wl1t, bl1c, wl2t, bl2c):
    nb = NP // 512
    return pl.pallas_call(
        _k7_body,
        grid=(nb,),
        in_specs=[pl.BlockSpec((C1, 512), lambda i: (0, i)),
                  pl.BlockSpec((C1, 512), lambda i: (0, i)),
                  pl.BlockSpec((1, 512), lambda i: (0, i)),
                  pl.BlockSpec((1, 512), lambda i: (0, i)),
                  pl.BlockSpec((C2, C1), lambda i: (0, 0)),
                  pl.BlockSpec((C2, 1), lambda i: (0, 0)),
                  pl.BlockSpec((100, C2), lambda i: (0, 0)),
                  pl.BlockSpec((100, 1), lambda i: (0, 0)),
                  pl.BlockSpec((1, 100), lambda i: (0, 0)),
                  pl.BlockSpec((1, 1), lambda i: (0, 0))],
        out_specs=pl.BlockSpec((1, G), lambda i: (0, 0)),
        out_shape=jax.ShapeDtypeStruct((1, G), jnp.float32),
        scratch_shapes=[pltpu.VMEM((C2, G), jnp.float32),
                        pltpu.VMEM((1, G), jnp.float32)],
    )(p2t, h1t, selfn, batchp, w2t, b2c, wl1t, bl1c, wl2t, bl2c)


@functools.cache
def _sc_kernels():
    mesh = _sc_mesh()
    cp = pltpu.CompilerParams(needs_layout_passes=False,
                              use_tc_tiling_on_sc=False)
    k1 = functools.partial(
        pl.kernel, mesh=mesh, compiler_params=cp,
        out_type=jax.ShapeDtypeStruct((NW, NP), jnp.float32),
        scratch_types=[
            pltpu.VMEM((CH4,), jnp.int32),
            pltpu.VMEM((CH4,), jnp.float32),
            pltpu.VMEM((NP,), jnp.float32),
        ],
    )(_k1_body)
    k3 = functools.partial(
        pl.kernel, mesh=mesh, compiler_params=cp,
        out_type=jax.ShapeDtypeStruct((EPAD,), jnp.float32),
        scratch_types=[
            pltpu.VMEM((CH4,), jnp.int32),
            pltpu.VMEM((CH4,), jnp.int32),
            pltpu.VMEM((CH4,), jnp.float32),
            pltpu.VMEM((CH4,), jnp.float32),
            pltpu.VMEM((NP,), jnp.float32),
        ],
    )(_k3_body)
    k4 = functools.partial(
        pl.kernel, mesh=mesh, compiler_params=cp,
        out_type=jax.ShapeDtypeStruct((3, NW, NP), jnp.float32),
        scratch_types=[
            pltpu.VMEM((CH4,), jnp.int32),
            pltpu.VMEM((CH4,), jnp.int32),
            pltpu.VMEM((CH4,), jnp.float32),
            pltpu.VMEM((NP,), jnp.float32),
            pltpu.VMEM((NP,), jnp.float32),
        ],
    )(_k4_body)
    k6 = functools.partial(
        pl.kernel, mesh=mesh, compiler_params=cp,
        out_type=jax.ShapeDtypeStruct((C1, NP), jnp.float32),
        scratch_types=[
            pltpu.VMEM((CH6,), jnp.int32),
            pltpu.VMEM((CH6,), jnp.int32),
            pltpu.VMEM((CH6,), jnp.float32),
            pltpu.VMEM((NP,), jnp.float32),
            pltpu.VMEM((NP,), jnp.float32),
        ],
    )(_k6_body)
    return k1, k3, k4, k6


def kernel(x, edge_index, edge_attr, batch, W1, b1, W2, b2, Wl1, bl1, Wl2, bl2):
    _k1_deg, _k3_norm, _k4_prop1, _k6_prop2 = _sc_kernels()
    f32 = jnp.float32
    pad_e = EPAD - E
    row = jnp.concatenate([edge_index[0], jnp.zeros((pad_e,), jnp.int32)])
    col = jnp.concatenate([edge_index[1], jnp.zeros((pad_e,), jnp.int32)])
    w = jnp.concatenate([edge_attr.astype(f32), jnp.zeros((pad_e,), f32)])

    xt = jnp.pad(x.astype(f32).T, ((0, 0), (0, NP - N)))
    batchp = jnp.pad(batch, (0, NP - N), constant_values=G).reshape(1, NP)
    zerosN = jnp.zeros((NP,), f32)

    w1t = jnp.pad(W1.T, ((0, C1 - 100), (0, 0)))
    b1c = jnp.pad(b1.reshape(100, 1), ((0, C1 - 100), (0, 0)))
    w2t = jnp.pad(W2.T, ((0, 0), (0, C1 - 100)))
    b2c = b2.reshape(C2, 1)
    wl1t = Wl1.T
    bl1c = bl1.reshape(100, 1)
    wl2t = Wl2.T
    bl2c = bl2.reshape(1, 1)

    part_deg = _k1_deg(col, w, zerosN)
    dinv2d, selfn = _k2_dinv(part_deg)
    norm = _k3_norm(row, col, w, dinv2d.reshape(NP))
    part1 = _k4_prop1(row, col, norm, xt, zerosN)
    h1t = _k5_h1(part1, xt, selfn, w1t, b1c)
    p2t = _k6_prop2(row, col, norm, h1t, zerosN)
    out = _k7_head(p2t, h1t, selfn, batchp, w2t, b2c,
                   wl1t, bl1c, wl2t, bl2c)
    return out[0]
